# baseline (speedup 1.0000x reference)
import functools

import jax
import jax.numpy as jnp
import numpy as np
from jax import lax
from jax.experimental import pallas as pl
from jax.experimental.pallas import tpu as pltpu
from jax.experimental.pallas import tpu_sc as plsc

_TAU = 0.9
_SC_ITERS = 28
_ROWS = 128
_N = 32768
_NW = 32
_RPW = _ROWS // _NW
_VPR = _N // 16


_GDN = lax.GatherDimensionNumbers(
    offset_dims=(), collapsed_slice_dims=(0,), start_index_map=(0,)
)


def _shuffle(v, perm):
    return lax.gather(
        v,
        perm[:, None],
        dimension_numbers=_GDN,
        slice_sizes=(1,),
        mode=lax.GatherScatterMode.PROMISE_IN_BOUNDS,
    )


def _allreduce(v, op):
    iota = lax.iota(jnp.int32, 16)
    for shift in (8, 4, 2, 1):
        v = op(v, _shuffle(v, (iota + shift) & 15))
    return v


def _row_kernel(z_hbm, out_hbm, z_v, e_v):
    wid = lax.axis_index("s") * 2 + lax.axis_index("c")

    def do_row(r, _):
        row = wid * _RPW + r
        pltpu.sync_copy(z_hbm.at[row], z_v)

        def maxbody(i, acc):
            return jnp.maximum(acc, z_v[pl.ds(i * 16, 16)])

        mvec = lax.fori_loop(
            0, _VPR, maxbody, jnp.full((16,), -jnp.inf, jnp.float32)
        )
        m = _allreduce(mvec, jnp.maximum)

        def expbody(i, acc):
            ev = jnp.exp(z_v[pl.ds(i * 16, 16)] - m)
            e_v[pl.ds(i * 16, 16)] = ev
            return acc + ev

        svec = lax.fori_loop(0, _VPR, expbody, jnp.zeros((16,), jnp.float32))
        s = _allreduce(svec, jnp.add)
        target = _TAU * s

        lo0 = lax.bitcast_convert_type(
            s * np.float32((1.0 - _TAU) / 65536.0), jnp.int32
        )
        hi0 = jnp.full((16,), 0x3F800001, jnp.int32)

        def bisect(_, carry):
            lo, hi = carry
            mid = (lo & hi) + ((lo ^ hi) >> 1)
            mid_f = lax.bitcast_convert_type(mid, jnp.float32)

            def gbody(i, acc):
                ev = e_v[pl.ds(i * 16, 16)]
                return acc + jnp.where(ev >= mid_f, ev, 0.0)

            gvec = lax.fori_loop(0, _VPR, gbody, jnp.zeros((16,), jnp.float32))
            g = _allreduce(gvec, jnp.add)
            pred = g >= target
            return jnp.where(pred, mid, lo), jnp.where(pred, hi, mid)

        lo, _hi = lax.fori_loop(0, _SC_ITERS, bisect, (lo0, hi0))
        lo_f = lax.bitcast_convert_type(lo, jnp.float32)

        def thbody(i, acc):
            ev = e_v[pl.ds(i * 16, 16)]
            return jnp.maximum(acc, jnp.where(ev <= lo_f, ev, 0.0))

        tvec = lax.fori_loop(0, _VPR, thbody, jnp.zeros((16,), jnp.float32))
        theta = _allreduce(tvec, jnp.maximum)

        def outbody(i, _):
            sl = pl.ds(i * 16, 16)
            ev = e_v[sl]
            zv = z_v[sl]
            z_v[sl] = jnp.where(ev >= theta, jnp.maximum(zv, 0.0), 0.0)
            return 0

        lax.fori_loop(0, _VPR, outbody, 0)
        pltpu.sync_copy(z_v, out_hbm.at[row])
        return 0

    lax.fori_loop(0, _RPW, do_row, 0)


@jax.jit
def kernel(z):
    mesh = plsc.VectorSubcoreMesh(core_axis_name="c", subcore_axis_name="s")
    f = pl.kernel(
        _row_kernel,
        mesh=mesh,
        out_type=jax.ShapeDtypeStruct((_ROWS, _N), jnp.float32),
        scratch_types=[
            pltpu.VMEM((_N,), jnp.float32),
            pltpu.VMEM((_N,), jnp.float32),
        ],
    )
    return f(z)


# speedup vs baseline: 13.0782x; 13.0782x over previous
import functools

import jax
import jax.numpy as jnp
import numpy as np
from jax.experimental import pallas as pl

_TAU = 0.9
_N_ITERS = 28


def _topk_mask_kernel(z_ref, out_ref):
    z = z_ref[:]
    m = jnp.max(z, axis=1, keepdims=True)
    e = jnp.exp(z - m)
    s = jnp.sum(e, axis=1, keepdims=True)
    target = _TAU * s

    lo = jax.lax.bitcast_convert_type(
        s * np.float32((1.0 - _TAU) / 65536.0), jnp.int32
    )
    hi = jnp.zeros_like(lo) + np.int32(0x3F800001)

    def body(_, carry):
        lo, hi = carry
        mid = (lo & hi) + ((lo ^ hi) >> 1)
        mid_f = jax.lax.bitcast_convert_type(mid, jnp.float32)
        g = jnp.sum(jnp.where(e >= mid_f, e, 0.0), axis=1, keepdims=True)
        pred = g >= target
        return jnp.where(pred, mid, lo), jnp.where(pred, hi, mid)

    lo, hi = jax.lax.fori_loop(0, _N_ITERS, body, (lo, hi))

    lo_f = jax.lax.bitcast_convert_type(lo, jnp.float32)
    theta = jnp.max(jnp.where(e <= lo_f, e, 0.0), axis=1, keepdims=True)
    out_ref[:] = jnp.where(e >= theta, jnp.maximum(z, 0.0), 0.0)


@jax.jit
def kernel(z):
    rows, n = z.shape
    block_rows = 64
    grid = (rows // block_rows,)
    return pl.pallas_call(
        _topk_mask_kernel,
        grid=grid,
        in_specs=[pl.BlockSpec((block_rows, n), lambda i: (i, 0))],
        out_specs=pl.BlockSpec((block_rows, n), lambda i: (i, 0)),
        out_shape=jax.ShapeDtypeStruct((rows, n), jnp.float32),
    )(z)
